# Initial kernel scaffold; baseline (speedup 1.0000x reference)
import jax
import jax.numpy as jnp
from jax.experimental import pallas as pl


def kernel(x_l, edge_index_l, edge_attr_l, x_r, edge_index_r, edge_attr_r, labels, params):
    raise NotImplementedError("write your pallas kernel here")


# speedup vs baseline: 2.9201x; 2.9201x over previous
import functools

import jax
import jax.numpy as jnp
from jax import lax
from jax.experimental import pallas as pl
from jax.experimental.pallas import tpu as pltpu
from jax.experimental.pallas import tpu_sc as plsc

N = 10000
E = 320000
DH = 128
WP = 384
L = 1024
NC = 2
NS = 16
EP = E // NS
RPT = N // NS
CH = 80
NCH = EP // CH
LPT = L // NS

_f32 = jnp.float32


def _sc_mesh():
    return plsc.VectorSubcoreMesh(
        core_axis_name="c", subcore_axis_name="s", num_cores=NC, num_subcores=NS
    )


def _hist_body(ei_ref, q_ref, z_ref, ones_ref, out_ref, src_v, dst_v, qrow_v,
               one_v, acc_sh):
    cid = lax.axis_index("c")
    sid = lax.axis_index("s")
    pl.when(sid == 0)(lambda: pltpu.sync_copy(z_ref, acc_sh))
    pltpu.sync_copy(ones_ref, one_v)
    plsc.subcore_barrier()
    ebase = cid * (2 * E) + sid * EP

    def chunk(j, carry):
        off = ebase + j * CH
        pltpu.sync_copy(ei_ref.at[pl.ds(off, CH)], src_v)
        pltpu.sync_copy(ei_ref.at[pl.ds(off + E, CH)], dst_v)
        pltpu.sync_copy(q_ref.at[cid, pl.ds(sid * EP + j * CH, CH)], qrow_v)
        pltpu.sync_copy(one_v, acc_sh.at[src_v], add=True)
        pltpu.sync_copy(qrow_v, acc_sh.at[dst_v], add=True)
        return carry

    lax.fori_loop(0, NCH, chunk, 0)
    plsc.subcore_barrier()
    pl.when(sid == 0)(lambda: pltpu.sync_copy(acc_sh, out_ref.at[cid]))


_hist = pl.kernel(
    _hist_body,
    out_type=jax.ShapeDtypeStruct((NC, N, 8), _f32),
    mesh=_sc_mesh(),
    scratch_types=[
        pltpu.VMEM((CH,), jnp.int32),
        pltpu.VMEM((CH,), jnp.int32),
        pltpu.VMEM((CH, 8), _f32),
        pltpu.VMEM((CH, 8), _f32),
        pltpu.VMEM_SHARED((N, 8), _f32),
    ],
)


def _seg_body(y_ref, ei_ref, base_ref, out_ref, src_v, dst_v, rows_v, acc_sh,
              sem):
    cid = lax.axis_index("c")
    sid = lax.axis_index("s")
    pl.when(sid == 0)(lambda: pltpu.sync_copy(base_ref.at[cid], acc_sh))
    plsc.subcore_barrier()
    ebase = cid * (2 * E) + sid * EP

    def chunk(j, carry):
        off = ebase + j * CH
        pltpu.sync_copy(ei_ref.at[pl.ds(off, CH)], src_v)
        pltpu.sync_copy(ei_ref.at[pl.ds(off + E, CH)], dst_v)
        pltpu.async_copy(y_ref.at[src_v], rows_v, sem).wait()
        pltpu.sync_copy(rows_v, acc_sh.at[dst_v], add=True)
        return carry

    lax.fori_loop(0, NCH, chunk, 0)
    plsc.subcore_barrier()
    pl.when(sid == 0)(lambda: pltpu.sync_copy(acc_sh, out_ref.at[cid]))


_segsum = pl.kernel(
    _seg_body,
    out_type=jax.ShapeDtypeStruct((NC, N, DH), _f32),
    mesh=_sc_mesh(),
    scratch_types=[
        pltpu.VMEM((CH,), jnp.int32),
        pltpu.VMEM((CH,), jnp.int32),
        pltpu.VMEM((CH, DH), _f32),
        pltpu.VMEM_SHARED((N, DH), _f32),
        pltpu.SemaphoreType.DMA,
    ],
)


def _gather_body(h_ref, labs_ref, out_ref, lab_v, rows_v, sem):
    cid = lax.axis_index("c")
    sid = lax.axis_index("s")
    pltpu.sync_copy(labs_ref.at[pl.ds(cid * L + sid * LPT, LPT)], lab_v)
    pltpu.async_copy(h_ref.at[lab_v], rows_v, sem).wait()
    pltpu.sync_copy(
        rows_v, out_ref.at[pl.ds(sid * LPT, LPT), pl.ds(cid * DH, DH)]
    )


_gather_merge = pl.kernel(
    _gather_body,
    out_type=jax.ShapeDtypeStruct((L, 2 * DH), _f32),
    mesh=_sc_mesh(),
    scratch_types=[
        pltpu.VMEM((LPT,), jnp.int32),
        pltpu.VMEM((LPT, DH), _f32),
        pltpu.SemaphoreType.DMA,
    ],
)


BM = 400


def _dense_kernel_body(x_ref, w_ref, b_ref, f_ref, s_ref, *, relu_in):
    x = x_ref[0]
    if relu_in:
        x = jnp.maximum(x, 0.0)
    res = jnp.dot(x, w_ref[0], preferred_element_type=_f32) + b_ref[0]
    f_ref[0] = res[:, 0:256]
    s_ref[0] = res[:, 256:264]


def _dense(xs, w, b, relu_in):
    return pl.pallas_call(
        functools.partial(_dense_kernel_body, relu_in=relu_in),
        grid=(2, N // BM),
        in_specs=[
            pl.BlockSpec((1, BM, DH), lambda g, i: (g, i, 0)),
            pl.BlockSpec((1, DH, WP), lambda g, i: (g, 0, 0)),
            pl.BlockSpec((1, 1, WP), lambda g, i: (g, 0, 0)),
        ],
        out_specs=[
            pl.BlockSpec((1, BM, 256), lambda g, i: (g, i, 0)),
            pl.BlockSpec((1, BM, 8), lambda g, i: (g, i, 0)),
        ],
        out_shape=[
            jax.ShapeDtypeStruct((2, N, 256), _f32),
            jax.ShapeDtypeStruct((2, N, 8), _f32),
        ],
    )(xs, w, b)


def _q_kernel_body(ea_ref, w_ref, b_ref, o_ref):
    o_ref[...] = (
        jnp.dot(ea_ref[...], w_ref[...], preferred_element_type=_f32)
        + b_ref[...]
    )


BE = 16000


def _edge_scalars(ea, w8, b8):
    return pl.pallas_call(
        _q_kernel_body,
        grid=(E // BE,),
        in_specs=[
            pl.BlockSpec((BE, 16), lambda i: (i, 0)),
            pl.BlockSpec((16, 8), lambda i: (0, 0)),
            pl.BlockSpec((1, 8), lambda i: (0, 0)),
        ],
        out_specs=pl.BlockSpec((BE, 8), lambda i: (i, 0)),
        out_shape=jax.ShapeDtypeStruct((E, 8), _f32),
    )(ea, w8, b8)


def _stats_kernel_body(s_ref, dc_ref, o_ref):
    s = s_ref[0][:, 0:3]
    l = jnp.where(s >= 0, s, 0.2 * s)
    m = jnp.max(l, axis=0, keepdims=True)
    ex = jnp.exp(l - m)
    deg = dc_ref[0][:, 0:1]
    z = jnp.sum(ex * (deg + 1.0), axis=0, keepdims=True)
    o_ref[0] = jnp.concatenate(
        [m, z, jnp.zeros((1, 2), _f32)], axis=1
    )


def _stats(s8, dc):
    return pl.pallas_call(
        _stats_kernel_body,
        grid=(2,),
        in_specs=[
            pl.BlockSpec((1, N, 8), lambda g: (g, 0, 0)),
            pl.BlockSpec((1, N, 8), lambda g: (g, 0, 0)),
        ],
        out_specs=pl.BlockSpec((1, 1, 8), lambda g: (g, 0, 0)),
        out_shape=jax.ShapeDtypeStruct((2, 1, 8), _f32),
    )(s8, dc)


BG = 2000


def _glue_kernel_body(f_ref, s_ref, dc_ref, st_ref, bias_ref, y_ref,
                      base_ref, *, ccol):
    s = s_ref[0][:, 0:3]
    l = jnp.where(s >= 0, s, 0.2 * s)
    st = st_ref[0]
    m = st[:, 0:3]
    z = st[:, 3:6]
    ex = jnp.exp(l - m)
    w3 = jnp.sum(ex / z, axis=1, keepdims=True) * (1.0 / 3.0)
    f = f_ref[0]
    xn = f[:, 128:256]
    y = xn * w3
    dc = dc_ref[0]
    c = dc[:, ccol:ccol + 1]
    base = f[:, 0:128] + y + bias_ref[0] + c * (1.0 / 20.0)
    y_ref[0] = y
    base_ref[0] = base


def _glue(f, s8, dc, bias, ccol):
    st = _stats(s8, dc)
    return pl.pallas_call(
        functools.partial(_glue_kernel_body, ccol=ccol),
        grid=(2, N // BG),
        in_specs=[
            pl.BlockSpec((1, BG, 256), lambda g, i: (g, i, 0)),
            pl.BlockSpec((1, BG, 8), lambda g, i: (g, i, 0)),
            pl.BlockSpec((1, BG, 8), lambda g, i: (g, i, 0)),
            pl.BlockSpec((1, 1, 8), lambda g, i: (g, 0, 0)),
            pl.BlockSpec((1, 1, DH), lambda g, i: (g, 0, 0)),
        ],
        out_specs=[
            pl.BlockSpec((1, BG, DH), lambda g, i: (g, i, 0)),
            pl.BlockSpec((1, BG, DH), lambda g, i: (g, i, 0)),
        ],
        out_shape=[
            jax.ShapeDtypeStruct((2, N, DH), _f32),
            jax.ShapeDtypeStruct((2, N, DH), _f32),
        ],
    )(f, s8, dc, st, bias)


def _fc_kernel_body(m_ref, w1_ref, b1_ref, w2_ref, b2_ref, o_ref):
    h = jnp.maximum(
        jnp.dot(m_ref[...], w1_ref[...], preferred_element_type=_f32)
        + b1_ref[...],
        0.0,
    )
    o_ref[...] = (
        jnp.dot(h, w2_ref[...], preferred_element_type=_f32) + b2_ref[...]
    )


def _fc(merged, w1, b1, w2, b2):
    return pl.pallas_call(
        _fc_kernel_body,
        out_shape=jax.ShapeDtypeStruct((L, 64), _f32),
    )(merged, w1, b1, w2, b2)


def _densify(p):
    wa = jnp.concatenate([p['Wa1'], p['Wa2'], p['Wa3']], 0)
    ba = jnp.concatenate([p['ba1'], p['ba2'], p['ba3']])
    wacomb = wa @ p['W_neigh']
    sb = wa @ p['b_neigh'] + ba
    din = p['W_self'].shape[1]
    wc = jnp.concatenate(
        [p['W_self'], p['W_neigh'], wacomb, jnp.zeros((WP - 259, din), _f32)],
        axis=0,
    )
    bv = jnp.concatenate(
        [p['b_self'], p['b_neigh'], sb, jnp.zeros((WP - 259,), _f32)]
    )
    return wc.T, bv


def kernel(x_l, edge_index_l, edge_attr_l, x_r, edge_index_r, edge_attr_r,
           labels, params):
    p = params

    w1l, b1l = _densify(p['conv1_l'])
    w1r, b1r = _densify(p['conv1_r'])
    w2l, b2l = _densify(p['conv2_l'])
    w2r, b2r = _densify(p['conv2_r'])
    w1 = jnp.stack([w1l, w1r])
    b1 = jnp.stack([b1l, b1r])[:, None, :]
    w2 = jnp.stack([w2l, w2r])
    b2 = jnp.stack([b2l, b2r])[:, None, :]
    def qself(pg):
        return jnp.sum(pg['W_edge']) + pg['b_edge'][0]

    def out_bias(pgl, pgr):
        return jnp.stack([
            pgl['bias'] + qself(pgl) * (1.0 / 20.0),
            pgr['bias'] + qself(pgr) * (1.0 / 20.0),
        ])[:, None, :]

    bias1 = out_bias(p['conv1_l'], p['conv1_r'])
    bias2 = out_bias(p['conv2_l'], p['conv2_r'])

    def edge_w(pg1, pg2):
        w8 = jnp.zeros((16, 8), _f32)
        w8 = w8.at[:, 1].set(pg1['W_edge'][0])
        w8 = w8.at[:, 2].set(pg2['W_edge'][0])
        b8 = jnp.zeros((1, 8), _f32)
        b8 = b8.at[0, 1].set(pg1['b_edge'][0])
        b8 = b8.at[0, 2].set(pg2['b_edge'][0])
        return w8, b8

    w8l, b8l = edge_w(p['conv1_l'], p['conv2_l'])
    w8r, b8r = edge_w(p['conv1_r'], p['conv2_r'])

    xs = jnp.stack([x_l, x_r])
    ei = jnp.concatenate([
        edge_index_l[0], edge_index_l[1],
        edge_index_r[0], edge_index_r[1],
    ])
    ei_g = jnp.concatenate([
        edge_index_l[0], edge_index_l[1],
        edge_index_r[0] + N, edge_index_r[1],
    ])
    labs = jnp.concatenate([labels[:, 0], labels[:, 1] + N])

    q_rows = jnp.stack([
        _edge_scalars(edge_attr_l, w8l, b8l),
        _edge_scalars(edge_attr_r, w8r, b8r),
    ])
    zrows = jnp.zeros((N, 8), _f32)
    onerows = jnp.zeros((CH, 8), _f32).at[:, 0].set(1.0)
    dc = _hist(ei, q_rows, zrows, onerows)

    f1, s1 = _dense(xs, w1, b1, relu_in=False)
    y1, base1 = _glue(f1, s1, dc, bias1, ccol=1)
    h1 = _segsum(y1.reshape(2 * N, DH), ei_g, base1)

    f2, s2 = _dense(h1, w2, b2, relu_in=True)
    y2, base2 = _glue(f2, s2, dc, bias2, ccol=2)
    h2 = _segsum(y2.reshape(2 * N, DH), ei_g, base2)

    merged = _gather_merge(h2.reshape(2 * N, DH), labs)
    return _fc(
        merged,
        p['fc1_W'].T, p['fc1_b'][None],
        p['fc2_W'].T, p['fc2_b'][None],
    )


# speedup vs baseline: 3.6270x; 1.2421x over previous
import functools

import jax
import jax.numpy as jnp
from jax import lax
from jax.experimental import pallas as pl
from jax.experimental.pallas import tpu as pltpu
from jax.experimental.pallas import tpu_sc as plsc

N = 10000
E = 320000
DH = 128
WP = 384
L = 1024
NC = 2
NS = 16
EP = E // NS
CH = 80
NCH = EP // CH
LPT = L // NS

_f32 = jnp.float32


def _sc_mesh():
    return plsc.VectorSubcoreMesh(
        core_axis_name="c", subcore_axis_name="s", num_cores=NC, num_subcores=NS
    )


def _hist_body(ei_ref, q_ref, z_ref, ones_ref, out_ref, src_v, dst_v, qrow_v,
               one_v, acc_sh):
    cid = lax.axis_index("c")
    sid = lax.axis_index("s")
    pl.when(sid == 0)(lambda: pltpu.sync_copy(z_ref, acc_sh))
    pltpu.sync_copy(ones_ref, one_v)
    plsc.subcore_barrier()
    ebase = cid * (2 * E) + sid * EP

    def chunk(j, carry):
        off = ebase + j * CH
        pltpu.sync_copy(ei_ref.at[pl.ds(off, CH)], src_v)
        pltpu.sync_copy(ei_ref.at[pl.ds(off + E, CH)], dst_v)
        pltpu.sync_copy(q_ref.at[cid, pl.ds(sid * EP + j * CH, CH)], qrow_v)
        pltpu.sync_copy(one_v, acc_sh.at[src_v], add=True)
        pltpu.sync_copy(qrow_v, acc_sh.at[dst_v], add=True)
        return carry

    lax.fori_loop(0, NCH, chunk, 0)
    plsc.subcore_barrier()
    pl.when(sid == 0)(lambda: pltpu.sync_copy(acc_sh, out_ref.at[cid]))


_hist = pl.kernel(
    _hist_body,
    out_type=jax.ShapeDtypeStruct((NC, N, 8), _f32),
    mesh=_sc_mesh(),
    scratch_types=[
        pltpu.VMEM((CH,), jnp.int32),
        pltpu.VMEM((CH,), jnp.int32),
        pltpu.VMEM((CH, 8), _f32),
        pltpu.VMEM((CH, 8), _f32),
        pltpu.VMEM_SHARED((N, 8), _f32),
    ],
)


def _seg_body(y_ref, ei_ref, base_ref, out_ref, src_v, dst_v, src_v1, dst_v1,
              rows_v, rows_v1, acc_sh, sem, sem1, semg, semg1):
    cid = lax.axis_index("c")
    sid = lax.axis_index("s")
    pl.when(sid == 0)(lambda: pltpu.sync_copy(base_ref.at[cid], acc_sh))
    plsc.subcore_barrier()
    ebase = cid * (2 * E) + sid * EP

    def pair(i, carry):
        off0 = ebase + (2 * i) * CH
        off1 = off0 + CH
        i0a = pltpu.async_copy(ei_ref.at[pl.ds(off0, CH)], src_v, sem)
        i0b = pltpu.async_copy(ei_ref.at[pl.ds(off0 + E, CH)], dst_v, sem)
        i1a = pltpu.async_copy(ei_ref.at[pl.ds(off1, CH)], src_v1, sem1)
        i1b = pltpu.async_copy(ei_ref.at[pl.ds(off1 + E, CH)], dst_v1, sem1)
        i0a.wait()
        i0b.wait()
        d0 = pltpu.async_copy(y_ref.at[src_v], rows_v, semg)
        i1a.wait()
        i1b.wait()
        d1 = pltpu.async_copy(y_ref.at[src_v1], rows_v1, semg1)
        d0.wait()
        pltpu.sync_copy(rows_v, acc_sh.at[dst_v], add=True)
        d1.wait()
        pltpu.sync_copy(rows_v1, acc_sh.at[dst_v1], add=True)
        return carry

    lax.fori_loop(0, NCH // 2, pair, 0)
    plsc.subcore_barrier()
    pl.when(sid == 0)(lambda: pltpu.sync_copy(acc_sh, out_ref.at[cid]))


_segsum = pl.kernel(
    _seg_body,
    out_type=jax.ShapeDtypeStruct((NC, N, DH), _f32),
    mesh=_sc_mesh(),
    scratch_types=[
        pltpu.VMEM((CH,), jnp.int32),
        pltpu.VMEM((CH,), jnp.int32),
        pltpu.VMEM((CH,), jnp.int32),
        pltpu.VMEM((CH,), jnp.int32),
        pltpu.VMEM((CH, DH), _f32),
        pltpu.VMEM((CH, DH), _f32),
        pltpu.VMEM_SHARED((N, DH), _f32),
        pltpu.SemaphoreType.DMA,
        pltpu.SemaphoreType.DMA,
        pltpu.SemaphoreType.DMA,
        pltpu.SemaphoreType.DMA,
    ],
)


def _gather_body(h_ref, labs_ref, out_ref, lab_v, rows_v, sem):
    cid = lax.axis_index("c")
    sid = lax.axis_index("s")
    pltpu.sync_copy(labs_ref.at[pl.ds(cid * L + sid * LPT, LPT)], lab_v)
    pltpu.async_copy(h_ref.at[lab_v], rows_v, sem).wait()
    pltpu.sync_copy(
        rows_v, out_ref.at[pl.ds(sid * LPT, LPT), pl.ds(cid * DH, DH)]
    )


_gather_merge = pl.kernel(
    _gather_body,
    out_type=jax.ShapeDtypeStruct((L, 2 * DH), _f32),
    mesh=_sc_mesh(),
    scratch_types=[
        pltpu.VMEM((LPT,), jnp.int32),
        pltpu.VMEM((LPT, DH), _f32),
        pltpu.SemaphoreType.DMA,
    ],
)


BM = 400


def _dense_kernel_body(x_ref, w_ref, b_ref, f_ref, s_ref, *, relu_in):
    x = x_ref[0]
    if relu_in:
        x = jnp.maximum(x, 0.0)
    res = jnp.dot(x, w_ref[0], preferred_element_type=_f32) + b_ref[0]
    f_ref[0] = res[:, 0:256]
    s_ref[0] = res[:, 256:264]


def _dense(xs, w, b, relu_in):
    return pl.pallas_call(
        functools.partial(_dense_kernel_body, relu_in=relu_in),
        grid=(2, N // BM),
        in_specs=[
            pl.BlockSpec((1, BM, DH), lambda g, i: (g, i, 0)),
            pl.BlockSpec((1, DH, WP), lambda g, i: (g, 0, 0)),
            pl.BlockSpec((1, 1, WP), lambda g, i: (g, 0, 0)),
        ],
        out_specs=[
            pl.BlockSpec((1, BM, 256), lambda g, i: (g, i, 0)),
            pl.BlockSpec((1, BM, 8), lambda g, i: (g, i, 0)),
        ],
        out_shape=[
            jax.ShapeDtypeStruct((2, N, 256), _f32),
            jax.ShapeDtypeStruct((2, N, 8), _f32),
        ],
    )(xs, w, b)


def _q_kernel_body(ea_ref, w_ref, b_ref, o_ref):
    o_ref[...] = (
        jnp.dot(ea_ref[...], w_ref[...], preferred_element_type=_f32)
        + b_ref[...]
    )


BE = 16000


def _edge_scalars(ea, w8, b8):
    return pl.pallas_call(
        _q_kernel_body,
        grid=(E // BE,),
        in_specs=[
            pl.BlockSpec((BE, 16), lambda i: (i, 0)),
            pl.BlockSpec((16, 8), lambda i: (0, 0)),
            pl.BlockSpec((1, 8), lambda i: (0, 0)),
        ],
        out_specs=pl.BlockSpec((BE, 8), lambda i: (i, 0)),
        out_shape=jax.ShapeDtypeStruct((E, 8), _f32),
    )(ea, w8, b8)


def _stats_kernel_body(s_ref, dc_ref, o_ref):
    s = s_ref[0][:, 0:3]
    l = jnp.where(s >= 0, s, 0.2 * s)
    m = jnp.max(l, axis=0, keepdims=True)
    ex = jnp.exp(l - m)
    deg = dc_ref[0][:, 0:1]
    z = jnp.sum(ex * (deg + 1.0), axis=0, keepdims=True)
    o_ref[0] = jnp.concatenate(
        [m, z, jnp.zeros((1, 2), _f32)], axis=1
    )


def _stats(s8, dc):
    return pl.pallas_call(
        _stats_kernel_body,
        grid=(2,),
        in_specs=[
            pl.BlockSpec((1, N, 8), lambda g: (g, 0, 0)),
            pl.BlockSpec((1, N, 8), lambda g: (g, 0, 0)),
        ],
        out_specs=pl.BlockSpec((1, 1, 8), lambda g: (g, 0, 0)),
        out_shape=jax.ShapeDtypeStruct((2, 1, 8), _f32),
    )(s8, dc)


BG = 2000


def _glue_kernel_body(f_ref, s_ref, dc_ref, st_ref, bias_ref,
                      y_ref, base_ref, *, ccol):
    s = s_ref[0][:, 0:3]
    l = jnp.where(s >= 0, s, 0.2 * s)
    st = st_ref[0]
    m = st[:, 0:3]
    z = st[:, 3:6]
    ex = jnp.exp(l - m)
    w3 = jnp.sum(ex / z, axis=1, keepdims=True) * (1.0 / 3.0)
    f = f_ref[0]
    xn = f[:, 128:256]
    y = xn * w3
    dc = dc_ref[0]
    c = dc[:, ccol:ccol + 1]
    base = f[:, 0:128] + y + bias_ref[0] + c * (1.0 / 20.0)
    y_ref[0] = y
    base_ref[0] = base


def _glue(f, s8, dc, bias, ccol):
    st = _stats(s8, dc)
    return pl.pallas_call(
        functools.partial(_glue_kernel_body, ccol=ccol),
        grid=(2, N // BG),
        in_specs=[
            pl.BlockSpec((1, BG, 256), lambda g, i: (g, i, 0)),
            pl.BlockSpec((1, BG, 8), lambda g, i: (g, i, 0)),
            pl.BlockSpec((1, BG, 8), lambda g, i: (g, i, 0)),
            pl.BlockSpec((1, 1, 8), lambda g, i: (g, 0, 0)),
            pl.BlockSpec((1, 1, DH), lambda g, i: (g, 0, 0)),
        ],
        out_specs=[
            pl.BlockSpec((1, BG, DH), lambda g, i: (g, i, 0)),
            pl.BlockSpec((1, BG, DH), lambda g, i: (g, i, 0)),
        ],
        out_shape=[
            jax.ShapeDtypeStruct((2, N, DH), _f32),
            jax.ShapeDtypeStruct((2, N, DH), _f32),
        ],
    )(f, s8, dc, st, bias)


def _fc_kernel_body(m_ref, w1_ref, b1_ref, w2_ref, b2_ref, o_ref):
    h = jnp.maximum(
        jnp.dot(m_ref[...], w1_ref[...], preferred_element_type=_f32)
        + b1_ref[...],
        0.0,
    )
    o_ref[...] = (
        jnp.dot(h, w2_ref[...], preferred_element_type=_f32) + b2_ref[...]
    )


def _fc(merged, w1, b1, w2, b2):
    return pl.pallas_call(
        _fc_kernel_body,
        out_shape=jax.ShapeDtypeStruct((L, 64), _f32),
    )(merged, w1, b1, w2, b2)


def _densify(p):
    wa = jnp.concatenate([p['Wa1'], p['Wa2'], p['Wa3']], 0)
    ba = jnp.concatenate([p['ba1'], p['ba2'], p['ba3']])
    wacomb = wa @ p['W_neigh']
    sb = wa @ p['b_neigh'] + ba
    din = p['W_self'].shape[1]
    wc = jnp.concatenate(
        [p['W_self'], p['W_neigh'], wacomb, jnp.zeros((WP - 259, din), _f32)],
        axis=0,
    )
    bv = jnp.concatenate(
        [p['b_self'], p['b_neigh'], sb, jnp.zeros((WP - 259,), _f32)]
    )
    return wc.T, bv


def kernel(x_l, edge_index_l, edge_attr_l, x_r, edge_index_r, edge_attr_r,
           labels, params):
    p = params

    w1l, b1l = _densify(p['conv1_l'])
    w1r, b1r = _densify(p['conv1_r'])
    w2l, b2l = _densify(p['conv2_l'])
    w2r, b2r = _densify(p['conv2_r'])
    w1 = jnp.stack([w1l, w1r])
    b1 = jnp.stack([b1l, b1r])[:, None, :]
    w2 = jnp.stack([w2l, w2r])
    b2 = jnp.stack([b2l, b2r])[:, None, :]
    def qself(pg):
        return jnp.sum(pg['W_edge']) + pg['b_edge'][0]

    def out_bias(pgl, pgr):
        return jnp.stack([
            pgl['bias'] + qself(pgl) * (1.0 / 20.0),
            pgr['bias'] + qself(pgr) * (1.0 / 20.0),
        ])[:, None, :]

    bias1 = out_bias(p['conv1_l'], p['conv1_r'])
    bias2 = out_bias(p['conv2_l'], p['conv2_r'])

    def edge_w(pg1, pg2):
        w8 = jnp.zeros((16, 8), _f32)
        w8 = w8.at[:, 1].set(pg1['W_edge'][0])
        w8 = w8.at[:, 2].set(pg2['W_edge'][0])
        b8 = jnp.zeros((1, 8), _f32)
        b8 = b8.at[0, 1].set(pg1['b_edge'][0])
        b8 = b8.at[0, 2].set(pg2['b_edge'][0])
        return w8, b8

    w8l, b8l = edge_w(p['conv1_l'], p['conv2_l'])
    w8r, b8r = edge_w(p['conv1_r'], p['conv2_r'])

    xs = jnp.stack([x_l, x_r])
    ei = jnp.concatenate([
        edge_index_l[0], edge_index_l[1],
        edge_index_r[0], edge_index_r[1],
    ])
    ei_g = jnp.concatenate([
        edge_index_l[0], edge_index_l[1],
        edge_index_r[0] + N, edge_index_r[1],
    ])
    labs = jnp.concatenate([labels[:, 0], labels[:, 1] + N])

    q_rows = jnp.stack([
        _edge_scalars(edge_attr_l, w8l, b8l),
        _edge_scalars(edge_attr_r, w8r, b8r),
    ])
    zrows = jnp.zeros((N, 8), _f32)
    onerows = jnp.zeros((CH, 8), _f32).at[:, 0].set(1.0)
    dc = _hist(ei, q_rows, zrows, onerows)

    f1, s1 = _dense(xs, w1, b1, relu_in=False)
    y1, base1 = _glue(f1, s1, dc, bias1, ccol=1)
    h1 = _segsum(y1.reshape(2 * N, DH), ei_g, base1)

    f2, s2 = _dense(h1, w2, b2, relu_in=True)
    y2, base2 = _glue(f2, s2, dc, bias2, ccol=2)
    h2 = _segsum(y2.reshape(2 * N, DH), ei_g, base2)

    merged = _gather_merge(h2.reshape(2 * N, DH), labs)
    return _fc(
        merged,
        p['fc1_W'].T, p['fc1_b'][None],
        p['fc2_W'].T, p['fc2_b'][None],
    )


# speedup vs baseline: 4.2801x; 1.1801x over previous
import functools

import jax
import jax.numpy as jnp
from jax import lax
from jax.experimental import pallas as pl
from jax.experimental.pallas import tpu as pltpu
from jax.experimental.pallas import tpu_sc as plsc

N = 10000
E = 320000
DH = 128
WP = 384
L = 1024
NC = 2
NS = 16
EP = E // NS
CH = 80
NCH = EP // CH
LPT = L // NS

_f32 = jnp.float32


def _sc_mesh():
    return plsc.VectorSubcoreMesh(
        core_axis_name="c", subcore_axis_name="s", num_cores=NC, num_subcores=NS
    )


def _hist_body(ei_ref, q_ref, z_ref, ones_ref, out_ref, src_v, dst_v, qrow_v,
               src_v1, dst_v1, qrow_v1, one_v, acc_sh, sem, sem1):
    cid = lax.axis_index("c")
    sid = lax.axis_index("s")
    pl.when(sid == 0)(lambda: pltpu.sync_copy(z_ref, acc_sh))
    pltpu.sync_copy(ones_ref, one_v)
    plsc.subcore_barrier()
    ebase = cid * (2 * E) + sid * EP

    def pair(i, carry):
        j0 = 2 * i
        off0 = ebase + j0 * CH
        off1 = off0 + CH
        q0 = sid * EP + j0 * CH
        d0a = pltpu.async_copy(ei_ref.at[pl.ds(off0, CH)], src_v, sem)
        d0b = pltpu.async_copy(ei_ref.at[pl.ds(off0 + E, CH)], dst_v, sem)
        d0c = pltpu.async_copy(q_ref.at[cid, pl.ds(q0, CH)], qrow_v, sem)
        d1a = pltpu.async_copy(ei_ref.at[pl.ds(off1, CH)], src_v1, sem1)
        d1b = pltpu.async_copy(ei_ref.at[pl.ds(off1 + E, CH)], dst_v1, sem1)
        d1c = pltpu.async_copy(q_ref.at[cid, pl.ds(q0 + CH, CH)], qrow_v1,
                               sem1)
        d0a.wait()
        d0b.wait()
        d0c.wait()
        pltpu.sync_copy(one_v, acc_sh.at[src_v], add=True)
        pltpu.sync_copy(qrow_v, acc_sh.at[dst_v], add=True)
        d1a.wait()
        d1b.wait()
        d1c.wait()
        pltpu.sync_copy(one_v, acc_sh.at[src_v1], add=True)
        pltpu.sync_copy(qrow_v1, acc_sh.at[dst_v1], add=True)
        return carry

    lax.fori_loop(0, NCH // 2, pair, 0)
    plsc.subcore_barrier()
    pl.when(sid == 0)(lambda: pltpu.sync_copy(acc_sh, out_ref.at[cid]))


_hist = pl.kernel(
    _hist_body,
    out_type=jax.ShapeDtypeStruct((NC, N, 8), _f32),
    mesh=_sc_mesh(),
    scratch_types=[
        pltpu.VMEM((CH,), jnp.int32),
        pltpu.VMEM((CH,), jnp.int32),
        pltpu.VMEM((CH, 8), _f32),
        pltpu.VMEM((CH,), jnp.int32),
        pltpu.VMEM((CH,), jnp.int32),
        pltpu.VMEM((CH, 8), _f32),
        pltpu.VMEM((CH, 8), _f32),
        pltpu.VMEM_SHARED((N, 8), _f32),
        pltpu.SemaphoreType.DMA,
        pltpu.SemaphoreType.DMA,
    ],
)


def _seg_body(y_ref, ei_ref, base_ref, out_ref, src_v, dst_v, src_v1, dst_v1,
              rows_v, rows_v1, acc_sh, sem, sem1, semg, semg1):
    cid = lax.axis_index("c")
    sid = lax.axis_index("s")
    pl.when(sid == 0)(lambda: pltpu.sync_copy(base_ref.at[cid], acc_sh))
    plsc.subcore_barrier()
    ebase = cid * (2 * E) + sid * EP

    def pair(i, carry):
        off0 = ebase + (2 * i) * CH
        off1 = off0 + CH
        i0a = pltpu.async_copy(ei_ref.at[pl.ds(off0, CH)], src_v, sem)
        i0b = pltpu.async_copy(ei_ref.at[pl.ds(off0 + E, CH)], dst_v, sem)
        i1a = pltpu.async_copy(ei_ref.at[pl.ds(off1, CH)], src_v1, sem1)
        i1b = pltpu.async_copy(ei_ref.at[pl.ds(off1 + E, CH)], dst_v1, sem1)
        i0a.wait()
        i0b.wait()
        d0 = pltpu.async_copy(y_ref.at[src_v], rows_v, semg)
        i1a.wait()
        i1b.wait()
        d1 = pltpu.async_copy(y_ref.at[src_v1], rows_v1, semg1)
        d0.wait()
        pltpu.sync_copy(rows_v, acc_sh.at[dst_v], add=True)
        d1.wait()
        pltpu.sync_copy(rows_v1, acc_sh.at[dst_v1], add=True)
        return carry

    lax.fori_loop(0, NCH // 2, pair, 0)
    plsc.subcore_barrier()
    pl.when(sid == 0)(lambda: pltpu.sync_copy(acc_sh, out_ref.at[cid]))


_segsum = pl.kernel(
    _seg_body,
    out_type=jax.ShapeDtypeStruct((NC, N, DH), _f32),
    mesh=_sc_mesh(),
    scratch_types=[
        pltpu.VMEM((CH,), jnp.int32),
        pltpu.VMEM((CH,), jnp.int32),
        pltpu.VMEM((CH,), jnp.int32),
        pltpu.VMEM((CH,), jnp.int32),
        pltpu.VMEM((CH, DH), _f32),
        pltpu.VMEM((CH, DH), _f32),
        pltpu.VMEM_SHARED((N, DH), _f32),
        pltpu.SemaphoreType.DMA,
        pltpu.SemaphoreType.DMA,
        pltpu.SemaphoreType.DMA,
        pltpu.SemaphoreType.DMA,
    ],
)


def _gather_body(h_ref, labs_ref, out_ref, lab_v, rows_v, sem):
    cid = lax.axis_index("c")
    sid = lax.axis_index("s")
    pltpu.sync_copy(labs_ref.at[pl.ds(cid * L + sid * LPT, LPT)], lab_v)
    pltpu.async_copy(h_ref.at[lab_v], rows_v, sem).wait()
    pltpu.sync_copy(
        rows_v, out_ref.at[pl.ds(sid * LPT, LPT), pl.ds(cid * DH, DH)]
    )


_gather_merge = pl.kernel(
    _gather_body,
    out_type=jax.ShapeDtypeStruct((L, 2 * DH), _f32),
    mesh=_sc_mesh(),
    scratch_types=[
        pltpu.VMEM((LPT,), jnp.int32),
        pltpu.VMEM((LPT, DH), _f32),
        pltpu.SemaphoreType.DMA,
    ],
)


BM = 400


def _dense_kernel_body(x_ref, w_ref, b_ref, f_ref, s_ref, *, relu_in):
    x = x_ref[0]
    if relu_in:
        x = jnp.maximum(x, 0.0)
    res = jnp.dot(x, w_ref[0], preferred_element_type=_f32) + b_ref[0]
    f_ref[0] = res[:, 0:256]
    s_ref[0] = res[:, 256:264]


def _dense(xs, w, b, relu_in):
    return pl.pallas_call(
        functools.partial(_dense_kernel_body, relu_in=relu_in),
        grid=(2, N // BM),
        in_specs=[
            pl.BlockSpec((1, BM, DH), lambda g, i: (g, i, 0)),
            pl.BlockSpec((1, DH, WP), lambda g, i: (g, 0, 0)),
            pl.BlockSpec((1, 1, WP), lambda g, i: (g, 0, 0)),
        ],
        out_specs=[
            pl.BlockSpec((1, BM, 256), lambda g, i: (g, i, 0)),
            pl.BlockSpec((1, BM, 8), lambda g, i: (g, i, 0)),
        ],
        out_shape=[
            jax.ShapeDtypeStruct((2, N, 256), _f32),
            jax.ShapeDtypeStruct((2, N, 8), _f32),
        ],
    )(xs, w, b)


def _q_kernel_body(ea_ref, w_ref, b_ref, o_ref):
    o_ref[...] = (
        jnp.dot(ea_ref[...], w_ref[...], preferred_element_type=_f32)
        + b_ref[...]
    )


BE = 16000


def _edge_scalars(ea, w8, b8):
    return pl.pallas_call(
        _q_kernel_body,
        grid=(E // BE,),
        in_specs=[
            pl.BlockSpec((BE, 16), lambda i: (i, 0)),
            pl.BlockSpec((16, 8), lambda i: (0, 0)),
            pl.BlockSpec((1, 8), lambda i: (0, 0)),
        ],
        out_specs=pl.BlockSpec((BE, 8), lambda i: (i, 0)),
        out_shape=jax.ShapeDtypeStruct((E, 8), _f32),
    )(ea, w8, b8)


def _stats_kernel_body(s_ref, dc_ref, o_ref):
    s = s_ref[0][:, 0:3]
    l = jnp.where(s >= 0, s, 0.2 * s)
    m = jnp.max(l, axis=0, keepdims=True)
    ex = jnp.exp(l - m)
    deg = dc_ref[0][:, 0:1]
    z = jnp.sum(ex * (deg + 1.0), axis=0, keepdims=True)
    o_ref[0] = jnp.concatenate(
        [m, z, jnp.zeros((1, 2), _f32)], axis=1
    )


def _stats(s8, dc):
    return pl.pallas_call(
        _stats_kernel_body,
        grid=(2,),
        in_specs=[
            pl.BlockSpec((1, N, 8), lambda g: (g, 0, 0)),
            pl.BlockSpec((1, N, 8), lambda g: (g, 0, 0)),
        ],
        out_specs=pl.BlockSpec((1, 1, 8), lambda g: (g, 0, 0)),
        out_shape=jax.ShapeDtypeStruct((2, 1, 8), _f32),
    )(s8, dc)


BG = 2000


def _glue_kernel_body(f_ref, s_ref, dc_ref, st_ref, bias_ref,
                      y_ref, base_ref, *, ccol):
    s = s_ref[0][:, 0:3]
    l = jnp.where(s >= 0, s, 0.2 * s)
    st = st_ref[0]
    m = st[:, 0:3]
    z = st[:, 3:6]
    ex = jnp.exp(l - m)
    w3 = jnp.sum(ex / z, axis=1, keepdims=True) * (1.0 / 3.0)
    f = f_ref[0]
    xn = f[:, 128:256]
    y = xn * w3
    dc = dc_ref[0]
    c = dc[:, ccol:ccol + 1]
    base = f[:, 0:128] + y + bias_ref[0] + c * (1.0 / 20.0)
    y_ref[0] = y
    base_ref[0] = base


def _glue(f, s8, dc, bias, ccol):
    st = _stats(s8, dc)
    return pl.pallas_call(
        functools.partial(_glue_kernel_body, ccol=ccol),
        grid=(2, N // BG),
        in_specs=[
            pl.BlockSpec((1, BG, 256), lambda g, i: (g, i, 0)),
            pl.BlockSpec((1, BG, 8), lambda g, i: (g, i, 0)),
            pl.BlockSpec((1, BG, 8), lambda g, i: (g, i, 0)),
            pl.BlockSpec((1, 1, 8), lambda g, i: (g, 0, 0)),
            pl.BlockSpec((1, 1, DH), lambda g, i: (g, 0, 0)),
        ],
        out_specs=[
            pl.BlockSpec((1, BG, DH), lambda g, i: (g, i, 0)),
            pl.BlockSpec((1, BG, DH), lambda g, i: (g, i, 0)),
        ],
        out_shape=[
            jax.ShapeDtypeStruct((2, N, DH), _f32),
            jax.ShapeDtypeStruct((2, N, DH), _f32),
        ],
    )(f, s8, dc, st, bias)


def _fc_kernel_body(m_ref, w1_ref, b1_ref, w2_ref, b2_ref, o_ref):
    h = jnp.maximum(
        jnp.dot(m_ref[...], w1_ref[...], preferred_element_type=_f32)
        + b1_ref[...],
        0.0,
    )
    o_ref[...] = (
        jnp.dot(h, w2_ref[...], preferred_element_type=_f32) + b2_ref[...]
    )


def _fc(merged, w1, b1, w2, b2):
    return pl.pallas_call(
        _fc_kernel_body,
        out_shape=jax.ShapeDtypeStruct((L, 64), _f32),
    )(merged, w1, b1, w2, b2)


def _densify(p):
    wa = jnp.concatenate([p['Wa1'], p['Wa2'], p['Wa3']], 0)
    ba = jnp.concatenate([p['ba1'], p['ba2'], p['ba3']])
    wacomb = wa @ p['W_neigh']
    sb = wa @ p['b_neigh'] + ba
    din = p['W_self'].shape[1]
    wc = jnp.concatenate(
        [p['W_self'], p['W_neigh'], wacomb, jnp.zeros((WP - 259, din), _f32)],
        axis=0,
    )
    bv = jnp.concatenate(
        [p['b_self'], p['b_neigh'], sb, jnp.zeros((WP - 259,), _f32)]
    )
    return wc.T, bv


def kernel(x_l, edge_index_l, edge_attr_l, x_r, edge_index_r, edge_attr_r,
           labels, params):
    p = params

    w1l, b1l = _densify(p['conv1_l'])
    w1r, b1r = _densify(p['conv1_r'])
    w2l, b2l = _densify(p['conv2_l'])
    w2r, b2r = _densify(p['conv2_r'])
    w1 = jnp.stack([w1l, w1r])
    b1 = jnp.stack([b1l, b1r])[:, None, :]
    w2 = jnp.stack([w2l, w2r])
    b2 = jnp.stack([b2l, b2r])[:, None, :]
    def qself(pg):
        return jnp.sum(pg['W_edge']) + pg['b_edge'][0]

    def out_bias(pgl, pgr):
        return jnp.stack([
            pgl['bias'] + qself(pgl) * (1.0 / 20.0),
            pgr['bias'] + qself(pgr) * (1.0 / 20.0),
        ])[:, None, :]

    bias1 = out_bias(p['conv1_l'], p['conv1_r'])
    bias2 = out_bias(p['conv2_l'], p['conv2_r'])

    def edge_w(pg1, pg2):
        w8 = jnp.zeros((16, 8), _f32)
        w8 = w8.at[:, 1].set(pg1['W_edge'][0])
        w8 = w8.at[:, 2].set(pg2['W_edge'][0])
        b8 = jnp.zeros((1, 8), _f32)
        b8 = b8.at[0, 1].set(pg1['b_edge'][0])
        b8 = b8.at[0, 2].set(pg2['b_edge'][0])
        return w8, b8

    w8l, b8l = edge_w(p['conv1_l'], p['conv2_l'])
    w8r, b8r = edge_w(p['conv1_r'], p['conv2_r'])

    xs = jnp.stack([x_l, x_r])
    ei = jnp.concatenate([
        edge_index_l[0], edge_index_l[1],
        edge_index_r[0], edge_index_r[1],
    ])
    ei_g = jnp.concatenate([
        edge_index_l[0], edge_index_l[1],
        edge_index_r[0] + N, edge_index_r[1],
    ])
    labs = jnp.concatenate([labels[:, 0], labels[:, 1] + N])

    q_rows = jnp.stack([
        _edge_scalars(edge_attr_l, w8l, b8l),
        _edge_scalars(edge_attr_r, w8r, b8r),
    ])
    zrows = jnp.zeros((N, 8), _f32)
    onerows = jnp.zeros((CH, 8), _f32).at[:, 0].set(1.0)
    dc = _hist(ei, q_rows, zrows, onerows)

    f1, s1 = _dense(xs, w1, b1, relu_in=False)
    y1, base1 = _glue(f1, s1, dc, bias1, ccol=1)
    h1 = _segsum(y1.reshape(2 * N, DH), ei_g, base1)

    f2, s2 = _dense(h1, w2, b2, relu_in=True)
    y2, base2 = _glue(f2, s2, dc, bias2, ccol=2)
    h2 = _segsum(y2.reshape(2 * N, DH), ei_g, base2)

    merged = _gather_merge(h2.reshape(2 * N, DH), labs)
    return _fc(
        merged,
        p['fc1_W'].T, p['fc1_b'][None],
        p['fc2_W'].T, p['fc2_b'][None],
    )


# speedup vs baseline: 4.3234x; 1.0101x over previous
import functools

import jax
import jax.numpy as jnp
from jax import lax
from jax.experimental import pallas as pl
from jax.experimental.pallas import tpu as pltpu
from jax.experimental.pallas import tpu_sc as plsc

N = 10000
E = 320000
DH = 128
WP = 384
L = 1024
NC = 2
NS = 16
EP = E // NS
CH = 80
NCH = EP // CH
LPT = L // NS

_f32 = jnp.float32


def _sc_mesh():
    return plsc.VectorSubcoreMesh(
        core_axis_name="c", subcore_axis_name="s", num_cores=NC, num_subcores=NS
    )


def _hist_body(ei_ref, q_ref, z_ref, ones_ref, out_ref, src_v, dst_v, qrow_v,
               src_v1, dst_v1, qrow_v1, one_v, acc_sh, sem, sem1):
    cid = lax.axis_index("c")
    sid = lax.axis_index("s")
    pl.when(sid == 0)(lambda: pltpu.sync_copy(z_ref, acc_sh))
    pltpu.sync_copy(ones_ref, one_v)
    plsc.subcore_barrier()
    ebase = cid * (2 * E) + sid * EP

    def pair(i, carry):
        j0 = 2 * i
        off0 = ebase + j0 * CH
        off1 = off0 + CH
        q0 = sid * EP + j0 * CH
        d0a = pltpu.async_copy(ei_ref.at[pl.ds(off0, CH)], src_v, sem)
        d0b = pltpu.async_copy(ei_ref.at[pl.ds(off0 + E, CH)], dst_v, sem)
        d0c = pltpu.async_copy(q_ref.at[cid, pl.ds(q0, CH)], qrow_v, sem)
        d1a = pltpu.async_copy(ei_ref.at[pl.ds(off1, CH)], src_v1, sem1)
        d1b = pltpu.async_copy(ei_ref.at[pl.ds(off1 + E, CH)], dst_v1, sem1)
        d1c = pltpu.async_copy(q_ref.at[cid, pl.ds(q0 + CH, CH)], qrow_v1,
                               sem1)
        d0a.wait()
        d0b.wait()
        d0c.wait()
        pltpu.sync_copy(one_v, acc_sh.at[src_v], add=True)
        pltpu.sync_copy(qrow_v, acc_sh.at[dst_v], add=True)
        d1a.wait()
        d1b.wait()
        d1c.wait()
        pltpu.sync_copy(one_v, acc_sh.at[src_v1], add=True)
        pltpu.sync_copy(qrow_v1, acc_sh.at[dst_v1], add=True)
        return carry

    lax.fori_loop(0, NCH // 2, pair, 0)
    plsc.subcore_barrier()
    pl.when(sid == 0)(lambda: pltpu.sync_copy(acc_sh, out_ref.at[cid]))


_hist = pl.kernel(
    _hist_body,
    out_type=jax.ShapeDtypeStruct((NC, N, 8), _f32),
    mesh=_sc_mesh(),
    scratch_types=[
        pltpu.VMEM((CH,), jnp.int32),
        pltpu.VMEM((CH,), jnp.int32),
        pltpu.VMEM((CH, 8), _f32),
        pltpu.VMEM((CH,), jnp.int32),
        pltpu.VMEM((CH,), jnp.int32),
        pltpu.VMEM((CH, 8), _f32),
        pltpu.VMEM((CH, 8), _f32),
        pltpu.VMEM_SHARED((N, 8), _f32),
        pltpu.SemaphoreType.DMA,
        pltpu.SemaphoreType.DMA,
    ],
)


def _seg_body(y_ref, ei_ref, base_ref, out_ref, src_v, dst_v, src_v1, dst_v1,
              rows_v, rows_v1, acc_sh, sem, sem1, semg, semg1):
    cid = lax.axis_index("c")
    sid = lax.axis_index("s")
    pl.when(sid == 0)(lambda: pltpu.sync_copy(base_ref.at[cid], acc_sh))
    plsc.subcore_barrier()
    ebase = cid * (2 * E) + sid * EP

    def pair(i, carry):
        off0 = ebase + (2 * i) * CH
        off1 = off0 + CH
        i0a = pltpu.async_copy(ei_ref.at[pl.ds(off0, CH)], src_v, sem)
        i0b = pltpu.async_copy(ei_ref.at[pl.ds(off0 + E, CH)], dst_v, sem)
        i1a = pltpu.async_copy(ei_ref.at[pl.ds(off1, CH)], src_v1, sem1)
        i1b = pltpu.async_copy(ei_ref.at[pl.ds(off1 + E, CH)], dst_v1, sem1)
        i0a.wait()
        i0b.wait()
        d0 = pltpu.async_copy(y_ref.at[src_v], rows_v, semg)
        i1a.wait()
        i1b.wait()
        d1 = pltpu.async_copy(y_ref.at[src_v1], rows_v1, semg1)
        d0.wait()
        s0 = pltpu.async_copy(rows_v, acc_sh.at[dst_v], semg, add=True)
        d1.wait()
        s1 = pltpu.async_copy(rows_v1, acc_sh.at[dst_v1], semg1, add=True)
        s0.wait()
        s1.wait()
        return carry

    lax.fori_loop(0, NCH // 2, pair, 0)
    plsc.subcore_barrier()
    pl.when(sid == 0)(lambda: pltpu.sync_copy(acc_sh, out_ref.at[cid]))


_segsum = pl.kernel(
    _seg_body,
    out_type=jax.ShapeDtypeStruct((NC, N, DH), _f32),
    mesh=_sc_mesh(),
    scratch_types=[
        pltpu.VMEM((CH,), jnp.int32),
        pltpu.VMEM((CH,), jnp.int32),
        pltpu.VMEM((CH,), jnp.int32),
        pltpu.VMEM((CH,), jnp.int32),
        pltpu.VMEM((CH, DH), _f32),
        pltpu.VMEM((CH, DH), _f32),
        pltpu.VMEM_SHARED((N, DH), _f32),
        pltpu.SemaphoreType.DMA,
        pltpu.SemaphoreType.DMA,
        pltpu.SemaphoreType.DMA,
        pltpu.SemaphoreType.DMA,
    ],
)


def _gather_body(h_ref, labs_ref, out_ref, lab_v, rows_v, sem):
    cid = lax.axis_index("c")
    sid = lax.axis_index("s")
    pltpu.sync_copy(labs_ref.at[pl.ds(cid * L + sid * LPT, LPT)], lab_v)
    pltpu.async_copy(h_ref.at[lab_v], rows_v, sem).wait()
    pltpu.sync_copy(
        rows_v, out_ref.at[pl.ds(sid * LPT, LPT), pl.ds(cid * DH, DH)]
    )


_gather_merge = pl.kernel(
    _gather_body,
    out_type=jax.ShapeDtypeStruct((L, 2 * DH), _f32),
    mesh=_sc_mesh(),
    scratch_types=[
        pltpu.VMEM((LPT,), jnp.int32),
        pltpu.VMEM((LPT, DH), _f32),
        pltpu.SemaphoreType.DMA,
    ],
)


BM = 400


def _dense_kernel_body(x_ref, w_ref, b_ref, f_ref, s_ref, *, relu_in):
    x = x_ref[0]
    if relu_in:
        x = jnp.maximum(x, 0.0)
    res = jnp.dot(x, w_ref[0], preferred_element_type=_f32) + b_ref[0]
    f_ref[0] = res[:, 0:256]
    s_ref[0] = res[:, 256:264]


def _dense(xs, w, b, relu_in):
    return pl.pallas_call(
        functools.partial(_dense_kernel_body, relu_in=relu_in),
        grid=(2, N // BM),
        in_specs=[
            pl.BlockSpec((1, BM, DH), lambda g, i: (g, i, 0)),
            pl.BlockSpec((1, DH, WP), lambda g, i: (g, 0, 0)),
            pl.BlockSpec((1, 1, WP), lambda g, i: (g, 0, 0)),
        ],
        out_specs=[
            pl.BlockSpec((1, BM, 256), lambda g, i: (g, i, 0)),
            pl.BlockSpec((1, BM, 8), lambda g, i: (g, i, 0)),
        ],
        out_shape=[
            jax.ShapeDtypeStruct((2, N, 256), _f32),
            jax.ShapeDtypeStruct((2, N, 8), _f32),
        ],
    )(xs, w, b)


def _q_kernel_body(ea_ref, w_ref, b_ref, o_ref):
    o_ref[...] = (
        jnp.dot(ea_ref[...], w_ref[...], preferred_element_type=_f32)
        + b_ref[...]
    )


BE = 16000


def _edge_scalars(ea, w8, b8):
    return pl.pallas_call(
        _q_kernel_body,
        grid=(E // BE,),
        in_specs=[
            pl.BlockSpec((BE, 16), lambda i: (i, 0)),
            pl.BlockSpec((16, 8), lambda i: (0, 0)),
            pl.BlockSpec((1, 8), lambda i: (0, 0)),
        ],
        out_specs=pl.BlockSpec((BE, 8), lambda i: (i, 0)),
        out_shape=jax.ShapeDtypeStruct((E, 8), _f32),
    )(ea, w8, b8)


def _stats_kernel_body(s_ref, dc_ref, o_ref):
    s = s_ref[0][:, 0:3]
    l = jnp.where(s >= 0, s, 0.2 * s)
    m = jnp.max(l, axis=0, keepdims=True)
    ex = jnp.exp(l - m)
    deg = dc_ref[0][:, 0:1]
    z = jnp.sum(ex * (deg + 1.0), axis=0, keepdims=True)
    o_ref[0] = jnp.concatenate(
        [m, z, jnp.zeros((1, 2), _f32)], axis=1
    )


def _stats(s8, dc):
    return pl.pallas_call(
        _stats_kernel_body,
        grid=(2,),
        in_specs=[
            pl.BlockSpec((1, N, 8), lambda g: (g, 0, 0)),
            pl.BlockSpec((1, N, 8), lambda g: (g, 0, 0)),
        ],
        out_specs=pl.BlockSpec((1, 1, 8), lambda g: (g, 0, 0)),
        out_shape=jax.ShapeDtypeStruct((2, 1, 8), _f32),
    )(s8, dc)


BG = 2000


def _glue_kernel_body(f_ref, s_ref, dc_ref, st_ref, bias_ref,
                      y_ref, base_ref, *, ccol):
    s = s_ref[0][:, 0:3]
    l = jnp.where(s >= 0, s, 0.2 * s)
    st = st_ref[0]
    m = st[:, 0:3]
    z = st[:, 3:6]
    ex = jnp.exp(l - m)
    w3 = jnp.sum(ex / z, axis=1, keepdims=True) * (1.0 / 3.0)
    f = f_ref[0]
    xn = f[:, 128:256]
    y = xn * w3
    dc = dc_ref[0]
    c = dc[:, ccol:ccol + 1]
    base = f[:, 0:128] + y + bias_ref[0] + c * (1.0 / 20.0)
    y_ref[0] = y
    base_ref[0] = base


def _glue(f, s8, dc, bias, ccol):
    st = _stats(s8, dc)
    return pl.pallas_call(
        functools.partial(_glue_kernel_body, ccol=ccol),
        grid=(2, N // BG),
        in_specs=[
            pl.BlockSpec((1, BG, 256), lambda g, i: (g, i, 0)),
            pl.BlockSpec((1, BG, 8), lambda g, i: (g, i, 0)),
            pl.BlockSpec((1, BG, 8), lambda g, i: (g, i, 0)),
            pl.BlockSpec((1, 1, 8), lambda g, i: (g, 0, 0)),
            pl.BlockSpec((1, 1, DH), lambda g, i: (g, 0, 0)),
        ],
        out_specs=[
            pl.BlockSpec((1, BG, DH), lambda g, i: (g, i, 0)),
            pl.BlockSpec((1, BG, DH), lambda g, i: (g, i, 0)),
        ],
        out_shape=[
            jax.ShapeDtypeStruct((2, N, DH), _f32),
            jax.ShapeDtypeStruct((2, N, DH), _f32),
        ],
    )(f, s8, dc, st, bias)


def _fc_kernel_body(m_ref, w1_ref, b1_ref, w2_ref, b2_ref, o_ref):
    h = jnp.maximum(
        jnp.dot(m_ref[...], w1_ref[...], preferred_element_type=_f32)
        + b1_ref[...],
        0.0,
    )
    o_ref[...] = (
        jnp.dot(h, w2_ref[...], preferred_element_type=_f32) + b2_ref[...]
    )


def _fc(merged, w1, b1, w2, b2):
    return pl.pallas_call(
        _fc_kernel_body,
        out_shape=jax.ShapeDtypeStruct((L, 64), _f32),
    )(merged, w1, b1, w2, b2)


def _densify(p):
    wa = jnp.concatenate([p['Wa1'], p['Wa2'], p['Wa3']], 0)
    ba = jnp.concatenate([p['ba1'], p['ba2'], p['ba3']])
    wacomb = wa @ p['W_neigh']
    sb = wa @ p['b_neigh'] + ba
    din = p['W_self'].shape[1]
    wc = jnp.concatenate(
        [p['W_self'], p['W_neigh'], wacomb, jnp.zeros((WP - 259, din), _f32)],
        axis=0,
    )
    bv = jnp.concatenate(
        [p['b_self'], p['b_neigh'], sb, jnp.zeros((WP - 259,), _f32)]
    )
    return wc.T, bv


def kernel(x_l, edge_index_l, edge_attr_l, x_r, edge_index_r, edge_attr_r,
           labels, params):
    p = params

    w1l, b1l = _densify(p['conv1_l'])
    w1r, b1r = _densify(p['conv1_r'])
    w2l, b2l = _densify(p['conv2_l'])
    w2r, b2r = _densify(p['conv2_r'])
    w1 = jnp.stack([w1l, w1r])
    b1 = jnp.stack([b1l, b1r])[:, None, :]
    w2 = jnp.stack([w2l, w2r])
    b2 = jnp.stack([b2l, b2r])[:, None, :]
    def qself(pg):
        return jnp.sum(pg['W_edge']) + pg['b_edge'][0]

    def out_bias(pgl, pgr):
        return jnp.stack([
            pgl['bias'] + qself(pgl) * (1.0 / 20.0),
            pgr['bias'] + qself(pgr) * (1.0 / 20.0),
        ])[:, None, :]

    bias1 = out_bias(p['conv1_l'], p['conv1_r'])
    bias2 = out_bias(p['conv2_l'], p['conv2_r'])

    def edge_w(pg1, pg2):
        w8 = jnp.zeros((16, 8), _f32)
        w8 = w8.at[:, 1].set(pg1['W_edge'][0])
        w8 = w8.at[:, 2].set(pg2['W_edge'][0])
        b8 = jnp.zeros((1, 8), _f32)
        b8 = b8.at[0, 1].set(pg1['b_edge'][0])
        b8 = b8.at[0, 2].set(pg2['b_edge'][0])
        return w8, b8

    w8l, b8l = edge_w(p['conv1_l'], p['conv2_l'])
    w8r, b8r = edge_w(p['conv1_r'], p['conv2_r'])

    xs = jnp.stack([x_l, x_r])
    ei = jnp.concatenate([
        edge_index_l[0], edge_index_l[1],
        edge_index_r[0], edge_index_r[1],
    ])
    ei_g = jnp.concatenate([
        edge_index_l[0], edge_index_l[1],
        edge_index_r[0] + N, edge_index_r[1],
    ])
    labs = jnp.concatenate([labels[:, 0], labels[:, 1] + N])

    q_rows = jnp.stack([
        _edge_scalars(edge_attr_l, w8l, b8l),
        _edge_scalars(edge_attr_r, w8r, b8r),
    ])
    zrows = jnp.zeros((N, 8), _f32)
    onerows = jnp.zeros((CH, 8), _f32).at[:, 0].set(1.0)
    dc = _hist(ei, q_rows, zrows, onerows)

    f1, s1 = _dense(xs, w1, b1, relu_in=False)
    y1, base1 = _glue(f1, s1, dc, bias1, ccol=1)
    h1 = _segsum(y1.reshape(2 * N, DH), ei_g, base1)

    f2, s2 = _dense(h1, w2, b2, relu_in=True)
    y2, base2 = _glue(f2, s2, dc, bias2, ccol=2)
    h2 = _segsum(y2.reshape(2 * N, DH), ei_g, base2)

    merged = _gather_merge(h2.reshape(2 * N, DH), labs)
    return _fc(
        merged,
        p['fc1_W'].T, p['fc1_b'][None],
        p['fc2_W'].T, p['fc2_b'][None],
    )


# speedup vs baseline: 4.6516x; 1.0759x over previous
import functools

import jax
import jax.numpy as jnp
from jax import lax
from jax.experimental import pallas as pl
from jax.experimental.pallas import tpu as pltpu
from jax.experimental.pallas import tpu_sc as plsc

N = 10000
E = 320000
DH = 128
WP = 384
L = 1024
NC = 2
NS = 16
EP = E // NS
CH = 80
NCH = EP // CH
LPT = L // NS

_f32 = jnp.float32


def _sc_mesh():
    return plsc.VectorSubcoreMesh(
        core_axis_name="c", subcore_axis_name="s", num_cores=NC, num_subcores=NS
    )


def _hist_body(ei_ref, q_ref, z_ref, ones_ref, out_ref, src_v, dst_v, qrow_v,
               src_v1, dst_v1, qrow_v1, one_v, acc_sh, sem, sem1):
    cid = lax.axis_index("c")
    sid = lax.axis_index("s")
    pl.when(sid == 0)(lambda: pltpu.sync_copy(z_ref, acc_sh))
    pltpu.sync_copy(ones_ref, one_v)
    plsc.subcore_barrier()
    ebase = cid * (2 * E) + sid * EP

    def pair(i, carry):
        j0 = 2 * i
        off0 = ebase + j0 * CH
        off1 = off0 + CH
        q0 = sid * EP + j0 * CH
        d0a = pltpu.async_copy(ei_ref.at[pl.ds(off0, CH)], src_v, sem)
        d0b = pltpu.async_copy(ei_ref.at[pl.ds(off0 + E, CH)], dst_v, sem)
        d0c = pltpu.async_copy(q_ref.at[cid, pl.ds(q0, CH)], qrow_v, sem)
        d1a = pltpu.async_copy(ei_ref.at[pl.ds(off1, CH)], src_v1, sem1)
        d1b = pltpu.async_copy(ei_ref.at[pl.ds(off1 + E, CH)], dst_v1, sem1)
        d1c = pltpu.async_copy(q_ref.at[cid, pl.ds(q0 + CH, CH)], qrow_v1,
                               sem1)
        d0a.wait()
        d0b.wait()
        d0c.wait()
        pltpu.sync_copy(one_v, acc_sh.at[src_v], add=True)
        pltpu.sync_copy(qrow_v, acc_sh.at[dst_v], add=True)
        d1a.wait()
        d1b.wait()
        d1c.wait()
        pltpu.sync_copy(one_v, acc_sh.at[src_v1], add=True)
        pltpu.sync_copy(qrow_v1, acc_sh.at[dst_v1], add=True)
        return carry

    lax.fori_loop(0, NCH // 2, pair, 0)
    plsc.subcore_barrier()
    pl.when(sid == 0)(lambda: pltpu.sync_copy(acc_sh, out_ref.at[cid]))


_hist = pl.kernel(
    _hist_body,
    out_type=jax.ShapeDtypeStruct((NC, N, 8), _f32),
    mesh=_sc_mesh(),
    scratch_types=[
        pltpu.VMEM((CH,), jnp.int32),
        pltpu.VMEM((CH,), jnp.int32),
        pltpu.VMEM((CH, 8), _f32),
        pltpu.VMEM((CH,), jnp.int32),
        pltpu.VMEM((CH,), jnp.int32),
        pltpu.VMEM((CH, 8), _f32),
        pltpu.VMEM((CH, 8), _f32),
        pltpu.VMEM_SHARED((N, 8), _f32),
        pltpu.SemaphoreType.DMA,
        pltpu.SemaphoreType.DMA,
    ],
)


def _seg_body(y_ref, ei_ref, base_ref, out_ref, src_v, dst_v, src_v1, dst_v1,
              src_v2, dst_v2, src_v3, dst_v3, rows_v, rows_v1, rows_v2,
              rows_v3, acc_sh, sem, sem1, sem2, sem3, semg, semg1, semg2,
              semg3):
    cid = lax.axis_index("c")
    sid = lax.axis_index("s")
    pl.when(sid == 0)(lambda: pltpu.sync_copy(base_ref.at[cid], acc_sh))
    plsc.subcore_barrier()
    ebase = cid * (2 * E) + sid * EP

    srcs = [src_v, src_v1, src_v2, src_v3]
    dsts = [dst_v, dst_v1, dst_v2, dst_v3]
    rows = [rows_v, rows_v1, rows_v2, rows_v3]
    isems = [sem, sem1, sem2, sem3]
    gsems = [semg, semg1, semg2, semg3]

    def sweep(start_chunk, carry, nb=4):
        off = ebase + start_chunk * CH
        iws = []
        for b in range(nb):
            iws.append(pltpu.async_copy(
                ei_ref.at[pl.ds(off + b * CH, CH)], srcs[b], isems[b]))
            iws.append(pltpu.async_copy(
                ei_ref.at[pl.ds(off + b * CH + E, CH)], dsts[b], isems[b]))
        gws = []
        for b in range(nb):
            iws[2 * b].wait()
            iws[2 * b + 1].wait()
            gws.append(pltpu.async_copy(y_ref.at[srcs[b]], rows[b], gsems[b]))
        sws = []
        for b in range(nb):
            gws[b].wait()
            sws.append(pltpu.async_copy(
                rows[b], acc_sh.at[dsts[b]], gsems[b], add=True))
        for b in range(nb):
            sws[b].wait()
        return carry

    lax.fori_loop(0, NCH // 4, lambda i, c: sweep(i * 4, c), 0)
    sweep((NCH // 4) * 4, 0, nb=2)
    plsc.subcore_barrier()
    pl.when(sid == 0)(lambda: pltpu.sync_copy(acc_sh, out_ref.at[cid]))


_segsum = pl.kernel(
    _seg_body,
    out_type=jax.ShapeDtypeStruct((NC, N, DH), _f32),
    mesh=_sc_mesh(),
    scratch_types=(
        [pltpu.VMEM((CH,), jnp.int32)] * 8
        + [pltpu.VMEM((CH, DH), _f32)] * 4
        + [pltpu.VMEM_SHARED((N, DH), _f32)]
        + [pltpu.SemaphoreType.DMA] * 8
    ),
)


def _gather_body(h_ref, labs_ref, out_ref, lab_v, rows_v, sem):
    cid = lax.axis_index("c")
    sid = lax.axis_index("s")
    pltpu.sync_copy(labs_ref.at[pl.ds(cid * L + sid * LPT, LPT)], lab_v)
    pltpu.async_copy(h_ref.at[lab_v], rows_v, sem).wait()
    pltpu.sync_copy(
        rows_v, out_ref.at[pl.ds(sid * LPT, LPT), pl.ds(cid * DH, DH)]
    )


_gather_merge = pl.kernel(
    _gather_body,
    out_type=jax.ShapeDtypeStruct((L, 2 * DH), _f32),
    mesh=_sc_mesh(),
    scratch_types=[
        pltpu.VMEM((LPT,), jnp.int32),
        pltpu.VMEM((LPT, DH), _f32),
        pltpu.SemaphoreType.DMA,
    ],
)


BM = 400


def _dense_kernel_body(x_ref, w_ref, b_ref, f_ref, s_ref, *, relu_in):
    x = x_ref[0]
    if relu_in:
        x = jnp.maximum(x, 0.0)
    res = jnp.dot(x, w_ref[0], preferred_element_type=_f32) + b_ref[0]
    f_ref[0] = res[:, 0:256]
    s_ref[0] = res[:, 256:264]


def _dense(xs, w, b, relu_in):
    return pl.pallas_call(
        functools.partial(_dense_kernel_body, relu_in=relu_in),
        grid=(2, N // BM),
        in_specs=[
            pl.BlockSpec((1, BM, DH), lambda g, i: (g, i, 0)),
            pl.BlockSpec((1, DH, WP), lambda g, i: (g, 0, 0)),
            pl.BlockSpec((1, 1, WP), lambda g, i: (g, 0, 0)),
        ],
        out_specs=[
            pl.BlockSpec((1, BM, 256), lambda g, i: (g, i, 0)),
            pl.BlockSpec((1, BM, 8), lambda g, i: (g, i, 0)),
        ],
        out_shape=[
            jax.ShapeDtypeStruct((2, N, 256), _f32),
            jax.ShapeDtypeStruct((2, N, 8), _f32),
        ],
    )(xs, w, b)


def _q_kernel_body(ea_ref, w_ref, b_ref, o_ref):
    o_ref[...] = (
        jnp.dot(ea_ref[...], w_ref[...], preferred_element_type=_f32)
        + b_ref[...]
    )


BE = 16000


def _edge_scalars(ea, w8, b8):
    return pl.pallas_call(
        _q_kernel_body,
        grid=(E // BE,),
        in_specs=[
            pl.BlockSpec((BE, 16), lambda i: (i, 0)),
            pl.BlockSpec((16, 8), lambda i: (0, 0)),
            pl.BlockSpec((1, 8), lambda i: (0, 0)),
        ],
        out_specs=pl.BlockSpec((BE, 8), lambda i: (i, 0)),
        out_shape=jax.ShapeDtypeStruct((E, 8), _f32),
    )(ea, w8, b8)


def _stats_kernel_body(s_ref, dc_ref, o_ref):
    s = s_ref[0][:, 0:3]
    l = jnp.where(s >= 0, s, 0.2 * s)
    m = jnp.max(l, axis=0, keepdims=True)
    ex = jnp.exp(l - m)
    deg = dc_ref[0][:, 0:1]
    z = jnp.sum(ex * (deg + 1.0), axis=0, keepdims=True)
    o_ref[0] = jnp.concatenate(
        [m, z, jnp.zeros((1, 2), _f32)], axis=1
    )


def _stats(s8, dc):
    return pl.pallas_call(
        _stats_kernel_body,
        grid=(2,),
        in_specs=[
            pl.BlockSpec((1, N, 8), lambda g: (g, 0, 0)),
            pl.BlockSpec((1, N, 8), lambda g: (g, 0, 0)),
        ],
        out_specs=pl.BlockSpec((1, 1, 8), lambda g: (g, 0, 0)),
        out_shape=jax.ShapeDtypeStruct((2, 1, 8), _f32),
    )(s8, dc)


BG = 2000


def _glue_kernel_body(f_ref, s_ref, dc_ref, st_ref, bias_ref,
                      y_ref, base_ref, *, ccol):
    s = s_ref[0][:, 0:3]
    l = jnp.where(s >= 0, s, 0.2 * s)
    st = st_ref[0]
    m = st[:, 0:3]
    z = st[:, 3:6]
    ex = jnp.exp(l - m)
    w3 = jnp.sum(ex / z, axis=1, keepdims=True) * (1.0 / 3.0)
    f = f_ref[0]
    xn = f[:, 128:256]
    y = xn * w3
    dc = dc_ref[0]
    c = dc[:, ccol:ccol + 1]
    base = f[:, 0:128] + y + bias_ref[0] + c * (1.0 / 20.0)
    y_ref[0] = y
    base_ref[0] = base


def _glue(f, s8, dc, bias, ccol):
    st = _stats(s8, dc)
    return pl.pallas_call(
        functools.partial(_glue_kernel_body, ccol=ccol),
        grid=(2, N // BG),
        in_specs=[
            pl.BlockSpec((1, BG, 256), lambda g, i: (g, i, 0)),
            pl.BlockSpec((1, BG, 8), lambda g, i: (g, i, 0)),
            pl.BlockSpec((1, BG, 8), lambda g, i: (g, i, 0)),
            pl.BlockSpec((1, 1, 8), lambda g, i: (g, 0, 0)),
            pl.BlockSpec((1, 1, DH), lambda g, i: (g, 0, 0)),
        ],
        out_specs=[
            pl.BlockSpec((1, BG, DH), lambda g, i: (g, i, 0)),
            pl.BlockSpec((1, BG, DH), lambda g, i: (g, i, 0)),
        ],
        out_shape=[
            jax.ShapeDtypeStruct((2, N, DH), _f32),
            jax.ShapeDtypeStruct((2, N, DH), _f32),
        ],
    )(f, s8, dc, st, bias)


def _fc_kernel_body(m_ref, w1_ref, b1_ref, w2_ref, b2_ref, o_ref):
    h = jnp.maximum(
        jnp.dot(m_ref[...], w1_ref[...], preferred_element_type=_f32)
        + b1_ref[...],
        0.0,
    )
    o_ref[...] = (
        jnp.dot(h, w2_ref[...], preferred_element_type=_f32) + b2_ref[...]
    )


def _fc(merged, w1, b1, w2, b2):
    return pl.pallas_call(
        _fc_kernel_body,
        out_shape=jax.ShapeDtypeStruct((L, 64), _f32),
    )(merged, w1, b1, w2, b2)


def _densify(p):
    wa = jnp.concatenate([p['Wa1'], p['Wa2'], p['Wa3']], 0)
    ba = jnp.concatenate([p['ba1'], p['ba2'], p['ba3']])
    wacomb = wa @ p['W_neigh']
    sb = wa @ p['b_neigh'] + ba
    din = p['W_self'].shape[1]
    wc = jnp.concatenate(
        [p['W_self'], p['W_neigh'], wacomb, jnp.zeros((WP - 259, din), _f32)],
        axis=0,
    )
    bv = jnp.concatenate(
        [p['b_self'], p['b_neigh'], sb, jnp.zeros((WP - 259,), _f32)]
    )
    return wc.T, bv


def kernel(x_l, edge_index_l, edge_attr_l, x_r, edge_index_r, edge_attr_r,
           labels, params):
    p = params

    w1l, b1l = _densify(p['conv1_l'])
    w1r, b1r = _densify(p['conv1_r'])
    w2l, b2l = _densify(p['conv2_l'])
    w2r, b2r = _densify(p['conv2_r'])
    w1 = jnp.stack([w1l, w1r])
    b1 = jnp.stack([b1l, b1r])[:, None, :]
    w2 = jnp.stack([w2l, w2r])
    b2 = jnp.stack([b2l, b2r])[:, None, :]
    def qself(pg):
        return jnp.sum(pg['W_edge']) + pg['b_edge'][0]

    def out_bias(pgl, pgr):
        return jnp.stack([
            pgl['bias'] + qself(pgl) * (1.0 / 20.0),
            pgr['bias'] + qself(pgr) * (1.0 / 20.0),
        ])[:, None, :]

    bias1 = out_bias(p['conv1_l'], p['conv1_r'])
    bias2 = out_bias(p['conv2_l'], p['conv2_r'])

    def edge_w(pg1, pg2):
        w8 = jnp.zeros((16, 8), _f32)
        w8 = w8.at[:, 1].set(pg1['W_edge'][0])
        w8 = w8.at[:, 2].set(pg2['W_edge'][0])
        b8 = jnp.zeros((1, 8), _f32)
        b8 = b8.at[0, 1].set(pg1['b_edge'][0])
        b8 = b8.at[0, 2].set(pg2['b_edge'][0])
        return w8, b8

    w8l, b8l = edge_w(p['conv1_l'], p['conv2_l'])
    w8r, b8r = edge_w(p['conv1_r'], p['conv2_r'])

    xs = jnp.stack([x_l, x_r])
    ei = jnp.concatenate([
        edge_index_l[0], edge_index_l[1],
        edge_index_r[0], edge_index_r[1],
    ])
    ei_g = jnp.concatenate([
        edge_index_l[0], edge_index_l[1],
        edge_index_r[0] + N, edge_index_r[1],
    ])
    labs = jnp.concatenate([labels[:, 0], labels[:, 1] + N])

    q_rows = jnp.stack([
        _edge_scalars(edge_attr_l, w8l, b8l),
        _edge_scalars(edge_attr_r, w8r, b8r),
    ])
    zrows = jnp.zeros((N, 8), _f32)
    onerows = jnp.zeros((CH, 8), _f32).at[:, 0].set(1.0)
    dc = _hist(ei, q_rows, zrows, onerows)

    f1, s1 = _dense(xs, w1, b1, relu_in=False)
    y1, base1 = _glue(f1, s1, dc, bias1, ccol=1)
    h1 = _segsum(y1.reshape(2 * N, DH), ei_g, base1)

    f2, s2 = _dense(h1, w2, b2, relu_in=True)
    y2, base2 = _glue(f2, s2, dc, bias2, ccol=2)
    h2 = _segsum(y2.reshape(2 * N, DH), ei_g, base2)

    merged = _gather_merge(h2.reshape(2 * N, DH), labs)
    return _fc(
        merged,
        p['fc1_W'].T, p['fc1_b'][None],
        p['fc2_W'].T, p['fc2_b'][None],
    )
